# baseline (speedup 1.0000x reference)
import functools

import jax
import jax.numpy as jnp
from jax import lax
from jax.experimental import pallas as pl
from jax.experimental.pallas import tpu as pltpu
from jax.experimental.pallas import tpu_sc as plsc

F32 = jnp.float32
I32 = jnp.int32

N0 = 10000
N1 = 160000
D = 128
N0P = 10240

NTILES = 16
NSC = 2
B = 128
KB = 2
SB = KB * B


def _mm_body(x_ref, w_ref, *out_refs, tanh):
  x = x_ref[...]
  if tanh:
    x = jnp.tanh(x)
  w = w_ref[...]
  for k, o in enumerate(out_refs):
    o[...] = jnp.dot(x, w[:, k * D:(k + 1) * D], preferred_element_type=F32,
                     precision=jax.lax.Precision.HIGHEST)


def _mm(x, wcat, tanh):
  n = x.shape[0]
  g = wcat.shape[1] // D
  bm = 640
  grid = (n // bm,)
  return pl.pallas_call(
      functools.partial(_mm_body, tanh=tanh),
      grid=grid,
      in_specs=[
          pl.BlockSpec((bm, D), lambda i: (i, 0)),
          pl.BlockSpec((D, g * D), lambda i: (0, 0)),
      ],
      out_specs=[pl.BlockSpec((bm, D), lambda i: (i, 0))] * g,
      out_shape=[jax.ShapeDtypeStruct((n, D), F32)] * g,
  )(x, wcat)


def _make_sc_spmm(out_rows, cr, counts, nbs):
  nch = out_rows // cr
  assert out_rows % cr == 0 and nch % 2 == 0
  tr = cr // NTILES
  per_tile = [c // NTILES for c in counts]
  nbs = [min(2000, p) for p in per_tile]
  for p, nb in zip(per_tile, nbs):
    assert p % nb == 0 and nb % 16 == 0

  mesh = plsc.VectorSubcoreMesh(core_axis_name="c", subcore_axis_name="s")

  def body(s0, s1, s2, s3,
           r0, c0, v0, r1, c1, v1, r2, c2, v2, r3, c3, v3,
           zeros_hbm, out_hbm,
           r_buf, c_buf, v_buf, loc_st, col_st, val_st,
           loc_idx, col_idx, val_t, gath, sem, accum):
    my_sc = lax.axis_index("c")
    my_tile = lax.axis_index("s")
    srcs = (s0, s1, s2, s3)
    nnz = ((r0, c0, v0), (r1, c1, v1), (r2, c2, v2), (r3, c3, v3))
    lane = jnp.arange(16, dtype=I32)

    def flush(src_ref, off):
      for sg in range(SB // 16):
        ids = lane + (sg * 16)
        m = ids < off
        sl = pl.ds(sg * 16, 16)
        ssl = pl.ds((sg % (B // 16)) * 16, 16)
        j = sg // (B // 16)
        col_idx[j, ssl] = jnp.where(m, col_st[sl], 0)
        loc_idx[j, ssl] = jnp.where(m, loc_st[sl], 0)
        val_t[sl] = jnp.where(m, val_st[sl], jnp.zeros((16,), F32))
      cps = [pltpu.async_copy(src_ref.at[col_idx.at[j]],
                              gath.at[pl.ds(j * B, B)], sem)
             for j in range(KB)]
      for cp in cps:
        cp.wait()
      def scale(i, _):
        v16 = val_t[pl.ds(jnp.bitwise_and(i, -16), 16)]
        vb = jnp.take_along_axis(
            v16, jnp.zeros((16,), I32) + jnp.bitwise_and(i, 15), axis=0)
        for j in range(D // 16):
          sl = pl.ds(j * 16, 16)
          gath[i, sl] = gath[i, sl] * vb
        return 0
      lax.fori_loop(0, SB, scale, 0)
      cps = [pltpu.async_copy(gath.at[pl.ds(j * B, B)],
                              accum.at[loc_idx.at[j]], sem, add=True)
             for j in range(KB)]
      for cp in cps:
        cp.wait()

    def do_chunk(i, _):
      chunk = i * 2 + my_sc
      lo = chunk * cr
      hi = lo + cr

      pltpu.sync_copy(zeros_hbm, accum.at[pl.ds(my_tile * tr, tr)])
      plsc.subcore_barrier()

      for t in range(4):
        nb = nbs[t]
        base = my_tile * per_tile[t]
        rows_h, cols_h, vals_h = nnz[t]

        def do_block(b, off, *, t=t, nb=nb, rows_h=rows_h, cols_h=cols_h,
                     vals_h=vals_h, base=base):
          st = base + b * nb
          cps = [pltpu.async_copy(rows_h.at[pl.ds(st, nb)],
                                  r_buf.at[pl.ds(0, nb)], sem),
                 pltpu.async_copy(cols_h.at[pl.ds(st, nb)],
                                  c_buf.at[pl.ds(0, nb)], sem),
                 pltpu.async_copy(vals_h.at[pl.ds(st, nb)],
                                  v_buf.at[pl.ds(0, nb)], sem)]
          for cp in cps:
            cp.wait()

          def group(g, off, *, t=t):
            sl = pl.ds(g * 16, 16)
            r = r_buf[sl]
            m = (r >= lo) & (r < hi)
            need_flush = (off + 16) > SB

            @pl.when(need_flush)
            def _():
              flush(srcs[t], off)
            off = jnp.where(need_flush, 0, off)

            s = m.astype(I32)
            for k2 in (1, 2, 4, 8):
              sh = jnp.take_along_axis(s, jnp.maximum(lane - k2, 0), axis=0)
              s = jnp.where(lane >= k2, s + sh, s)
            pos = jnp.where(m, off + s - m.astype(I32), SB + lane)
            plsc.store_scatter(loc_st, [pos], r - lo)
            plsc.store_scatter(col_st, [pos], c_buf[sl])
            plsc.store_scatter(val_st, [pos], v_buf[sl])
            return off + s[15]

          return lax.fori_loop(0, nb // 16, group, off)

        off = lax.fori_loop(0, per_tile[t] // nb, do_block, jnp.int32(0))

        @pl.when(off > 0)
        def _(*, t=t, off=off):
          flush(srcs[t], off)

      plsc.subcore_barrier()
      pltpu.sync_copy(
          accum.at[pl.ds(my_tile * tr, tr)],
          out_hbm.at[pl.ds(lo + my_tile * tr, tr)])
      plsc.subcore_barrier()
      return 0

    lax.fori_loop(0, nch // 2, do_chunk, 0)

  nbmax = max(nbs)
  return pl.kernel(
      body,
      out_type=jax.ShapeDtypeStruct((out_rows, D), F32),
      mesh=mesh,
      compiler_params=pltpu.CompilerParams(needs_layout_passes=False),
      scratch_types=[
          pltpu.VMEM((nbmax,), I32),
          pltpu.VMEM((nbmax,), I32),
          pltpu.VMEM((nbmax,), F32),
          pltpu.VMEM((SB + 16,), I32),
          pltpu.VMEM((SB + 16,), I32),
          pltpu.VMEM((SB + 16,), F32),
          pltpu.VMEM((KB, B), I32),
          pltpu.VMEM((KB, B), I32),
          pltpu.VMEM((SB,), F32),
          pltpu.VMEM((SB, D), F32),
          pltpu.SemaphoreType.DMA,
          pltpu.VMEM_SHARED((cr, D), F32),
      ],
  )


_SC_H0 = _make_sc_spmm(N0P, 5120, (160000, 160000, 10240, 320000),
                       (2000, 2000, 640, 2000))
N1P = 184320
_SC_H1 = _make_sc_spmm(N1P, 10240, (320000, 320000, 320000, 320000),
                       (2000, 2000, 2000, 2000))


def kernel(X0, X1, L0_rows, L0_cols, L0_vals, Lu0_rows, Lu0_cols, Lu0_vals,
           Ld0_rows, Ld0_cols, Ld0_vals, L1_rows, L1_cols, L1_vals,
           Lu1_rows, Lu1_cols, Lu1_vals, Ld1_rows, Ld1_cols, Ld1_vals,
           B1_rows, B1_cols, B1_vals,
           W1_L0, W1_U0, W1_D0, W1_B0, W1_L1, W1_U1, W1_D1, W1_B1,
           W2_L0, W2_U0, W2_D0, W2_B0, W2_L1, W2_U1, W2_D1, W2_B1,
           W3_L0, W3_U0, W3_D0, W3_B0, W3_L1, W3_U1, W3_D1, W3_B1,
           fc_W):
  W = {1: (W1_L0, W1_U0, W1_D0, W1_B0, W1_L1, W1_U1, W1_D1, W1_B1),
       2: (W2_L0, W2_U0, W2_D0, W2_B0, W2_L1, W2_U1, W2_D1, W2_B1),
       3: (W3_L0, W3_U0, W3_D0, W3_B0, W3_L1, W3_U1, W3_D1, W3_B1)}

  pad = 10240 - 10000
  ld0_r = jnp.pad(Ld0_rows, (0, pad))
  ld0_c = jnp.pad(Ld0_cols, (0, pad))
  ld0_v = jnp.pad(Ld0_vals, (0, pad))

  act0 = jnp.pad(X0, ((0, N0P - N0), (0, 0)))
  act1 = X1
  z0 = jnp.zeros((5120 // NTILES, D), F32)
  z1 = jnp.zeros((10240 // NTILES, D), F32)
  for l in (1, 2, 3)[:globals().get("_LAYERS", 3)]:
    w_l0, w_u0, w_d0, w_b0, w_l1, w_u1, w_d1, w_b1 = W[l]
    wc0 = jnp.concatenate([w_l0, w_u0, w_d0, w_b1], axis=1)
    wc1 = jnp.concatenate([w_l1, w_u1, w_d1, w_b0], axis=1)
    y0l, y0u, y0d, y0b = _mm(act0, wc0, tanh=(l > 1))
    y1l, y1u, y1d, y1b = _mm(act1, wc1, tanh=(l > 1))
    h0 = _SC_H0(y0l, y0u, y0d, y1b,
                L0_rows, L0_cols, L0_vals,
                Lu0_rows, Lu0_cols, Lu0_vals,
                ld0_r, ld0_c, ld0_v,
                B1_rows, B1_cols, B1_vals, z0)
    h1 = _SC_H1(y1l, y1u, y1d, y0b,
                L1_rows, L1_cols, L1_vals,
                Lu1_rows, Lu1_cols, Lu1_vals,
                Ld1_rows, Ld1_cols, Ld1_vals,
                B1_cols, B1_rows, B1_vals, z1)
    act0, act1 = h0, h1

  out0 = _mm(act0, fc_W, tanh=True)[0][:N0]
  out1 = _mm(act1, fc_W, tanh=True)[0][:N1]
  return out0, out1


# speedup vs baseline: 1.4136x; 1.4136x over previous
import functools

import jax
import jax.numpy as jnp
from jax import lax
from jax.experimental import pallas as pl
from jax.experimental.pallas import tpu as pltpu
from jax.experimental.pallas import tpu_sc as plsc

F32 = jnp.float32
I32 = jnp.int32

N0 = 10000
N1 = 160000
D = 128
N0P = 10240

NTILES = 16
NSC = 2
B = 128
KB = 2
SB = KB * B


def _mm_body(x_ref, w_ref, *out_refs, tanh):
  x = x_ref[...]
  if tanh:
    x = jnp.tanh(x)
  w = w_ref[...]
  for k, o in enumerate(out_refs):
    o[...] = jnp.dot(x, w[:, k * D:(k + 1) * D], preferred_element_type=F32,
                     precision=jax.lax.Precision.HIGHEST)


def _mm(x, wcat, tanh):
  n = x.shape[0]
  g = wcat.shape[1] // D
  bm = 640
  grid = (n // bm,)
  return pl.pallas_call(
      functools.partial(_mm_body, tanh=tanh),
      grid=grid,
      in_specs=[
          pl.BlockSpec((bm, D), lambda i: (i, 0)),
          pl.BlockSpec((D, g * D), lambda i: (0, 0)),
      ],
      out_specs=[pl.BlockSpec((bm, D), lambda i: (i, 0))] * g,
      out_shape=[jax.ShapeDtypeStruct((n, D), F32)] * g,
  )(x, wcat)


def _make_sc_spmm(out_rows, cr, counts, nbs, variant=0):
  nch = out_rows // cr
  assert out_rows % cr == 0 and nch % 2 == 0
  tr = cr // NTILES
  per_tile = [c // NTILES for c in counts]
  nbs = [min(2000, p) for p in per_tile]
  for p, nb in zip(per_tile, nbs):
    assert p % nb == 0 and nb % 16 == 0

  mesh = plsc.VectorSubcoreMesh(core_axis_name="c", subcore_axis_name="s")

  def body(s0, s1, s2, s3,
           r0, c0, v0, r1, c1, v1, r2, c2, v2, r3, c3, v3,
           zeros_hbm, out_hbm,
           r_buf, c_buf, v_buf, loc_st, col_st, val_st,
           loc_idx, col_idx, val_t, gath, sem_g, sem_s, sem_n, accum):
    my_sc = lax.axis_index("c")
    my_tile = lax.axis_index("s")
    srcs = (s0, s1, s2, s3)
    nnz = ((r0, c0, v0), (r1, c1, v1), (r2, c2, v2), (r3, c3, v3))
    lane = jnp.arange(16, dtype=I32)

    def fire_gather(src_ref, j):
      for sg in range(B // 16):
        ssl = pl.ds(sg * 16, 16)
        sl = pl.ds(j * B + sg * 16, 16)
        col_idx[j, ssl] = col_st[sl]
        loc_idx[j, ssl] = loc_st[sl]
        val_t[sl] = val_st[sl]
      pltpu.async_copy(src_ref.at[col_idx.at[j]],
                       gath.at[pl.ds(j * B, B)], sem_g)

    def fire_gather_partial(src_ref, j, off):
      for sg in range(B // 16):
        ids = lane + (j * B + sg * 16)
        m = ids < off
        sl = pl.ds(j * B + sg * 16, 16)
        ssl = pl.ds(sg * 16, 16)
        col_idx[j, ssl] = jnp.where(m, col_st[sl], 0)
        loc_idx[j, ssl] = jnp.where(m, loc_st[sl], 0)
        val_t[sl] = jnp.where(m, val_st[sl], jnp.zeros((16,), F32))
      pltpu.async_copy(src_ref.at[col_idx.at[j]],
                       gath.at[pl.ds(j * B, B)], sem_g)

    def flush(src_ref, off):
      @pl.when(off < B)
      def _():
        fire_gather_partial(src_ref, 0, off)

      @pl.when((off > B) & (off < SB))
      def _():
        fire_gather_partial(src_ref, 1, off)

      pltpu.make_async_copy(src_ref.at[col_idx.at[0]],
                            gath.at[pl.ds(0, B)], sem_g).wait()

      @pl.when(off > B)
      def _():
        pltpu.make_async_copy(src_ref.at[col_idx.at[1]],
                              gath.at[pl.ds(B, B)], sem_g).wait()

      def scale(i, _):
        v16 = val_t[pl.ds(jnp.bitwise_and(i, -16), 16)]
        vb = jnp.take_along_axis(
            v16, jnp.zeros((16,), I32) + jnp.bitwise_and(i, 15), axis=0)
        for j in range(D // 16):
          sl = pl.ds(j * 16, 16)
          gath[i, sl] = gath[i, sl] * vb
        return 0
      lax.fori_loop(0, SB, scale, 0)

      cp0 = pltpu.async_copy(gath.at[pl.ds(0, B)],
                             accum.at[loc_idx.at[0]], sem_s, add=True)

      @pl.when(off > B)
      def _():
        pltpu.async_copy(gath.at[pl.ds(B, B)],
                         accum.at[loc_idx.at[1]], sem_s, add=True)
      cp0.wait()

      @pl.when(off > B)
      def _():
        pltpu.make_async_copy(gath.at[pl.ds(B, B)],
                              accum.at[loc_idx.at[1]], sem_s).wait()

    def do_chunk(i, _):
      chunk = i * 2 + my_sc
      lo = chunk * cr
      hi = lo + cr

      pltpu.sync_copy(zeros_hbm, accum.at[pl.ds(my_tile * tr, tr)])
      plsc.subcore_barrier()

      for t in range(4):
        nb = nbs[t]
        base = my_tile * per_tile[t]
        rows_h, cols_h, vals_h = nnz[t]

        def do_block(b, off, *, t=t, nb=nb, rows_h=rows_h, cols_h=cols_h,
                     vals_h=vals_h, base=base):
          st = base + b * nb
          cps = [pltpu.async_copy(rows_h.at[pl.ds(st, nb)],
                                  r_buf.at[pl.ds(0, nb)], sem_n),
                 pltpu.async_copy(cols_h.at[pl.ds(st, nb)],
                                  c_buf.at[pl.ds(0, nb)], sem_n),
                 pltpu.async_copy(vals_h.at[pl.ds(st, nb)],
                                  v_buf.at[pl.ds(0, nb)], sem_n)]
          for cp in cps:
            cp.wait()

          def group(g, off, *, t=t):
            sl = pl.ds(g * 16, 16)
            r = r_buf[sl]
            m = (r >= lo) & (r < hi)
            need_flush = (off + 16) > SB

            @pl.when(need_flush)
            def _():
              flush(srcs[t], off)
            off = jnp.where(need_flush, 0, off)

            s = m.astype(I32)
            if not (variant & 2):
              for k2 in (1, 2, 4, 8):
                sh = jnp.take_along_axis(s, jnp.maximum(lane - k2, 0), axis=0)
                s = jnp.where(lane >= k2, s + sh, s)
            pos = jnp.where(m, off + s - m.astype(I32), SB + lane)
            if not (variant & 1):
              plsc.store_scatter(loc_st, [pos], r - lo)
              plsc.store_scatter(col_st, [pos], c_buf[sl])
              plsc.store_scatter(val_st, [pos], v_buf[sl])
            if variant & 4:
              new_off = off + 1
            else:
              new_off = off + s[15]

            @pl.when((off < B) & (new_off >= B))
            def _():
              fire_gather(srcs[t], 0)

            @pl.when((off < SB) & (new_off >= SB))
            def _():
              fire_gather(srcs[t], 1)
            return new_off

          return lax.fori_loop(0, nb // 16, group, off)

        off = lax.fori_loop(0, per_tile[t] // nb, do_block, jnp.int32(0))

        @pl.when(off > 0)
        def _(*, t=t, off=off):
          flush(srcs[t], off)

      plsc.subcore_barrier()
      pltpu.sync_copy(
          accum.at[pl.ds(my_tile * tr, tr)],
          out_hbm.at[pl.ds(lo + my_tile * tr, tr)])
      plsc.subcore_barrier()
      return 0

    lax.fori_loop(0, nch // 2, do_chunk, 0)

  nbmax = max(nbs)
  return pl.kernel(
      body,
      out_type=jax.ShapeDtypeStruct((out_rows, D), F32),
      mesh=mesh,
      compiler_params=pltpu.CompilerParams(needs_layout_passes=False),
      scratch_types=[
          pltpu.VMEM((nbmax,), I32),
          pltpu.VMEM((nbmax,), I32),
          pltpu.VMEM((nbmax,), F32),
          pltpu.VMEM((SB + 16,), I32),
          pltpu.VMEM((SB + 16,), I32),
          pltpu.VMEM((SB + 16,), F32),
          pltpu.VMEM((KB, B), I32),
          pltpu.VMEM((KB, B), I32),
          pltpu.VMEM((SB,), F32),
          pltpu.VMEM((SB, D), F32),
          pltpu.SemaphoreType.DMA,
          pltpu.SemaphoreType.DMA,
          pltpu.SemaphoreType.DMA,
          pltpu.VMEM_SHARED((cr, D), F32),
      ],
  )


_SC_H0 = _make_sc_spmm(N0P, 5120, (160000, 160000, 10240, 320000),
                       (2000, 2000, 640, 2000))
N1P = 184320
_SC_H1 = _make_sc_spmm(N1P, 10240, (320000, 320000, 320000, 320000),
                       (2000, 2000, 2000, 2000))


def kernel(X0, X1, L0_rows, L0_cols, L0_vals, Lu0_rows, Lu0_cols, Lu0_vals,
           Ld0_rows, Ld0_cols, Ld0_vals, L1_rows, L1_cols, L1_vals,
           Lu1_rows, Lu1_cols, Lu1_vals, Ld1_rows, Ld1_cols, Ld1_vals,
           B1_rows, B1_cols, B1_vals,
           W1_L0, W1_U0, W1_D0, W1_B0, W1_L1, W1_U1, W1_D1, W1_B1,
           W2_L0, W2_U0, W2_D0, W2_B0, W2_L1, W2_U1, W2_D1, W2_B1,
           W3_L0, W3_U0, W3_D0, W3_B0, W3_L1, W3_U1, W3_D1, W3_B1,
           fc_W):
  W = {1: (W1_L0, W1_U0, W1_D0, W1_B0, W1_L1, W1_U1, W1_D1, W1_B1),
       2: (W2_L0, W2_U0, W2_D0, W2_B0, W2_L1, W2_U1, W2_D1, W2_B1),
       3: (W3_L0, W3_U0, W3_D0, W3_B0, W3_L1, W3_U1, W3_D1, W3_B1)}

  pad = 10240 - 10000
  ld0_r = jnp.pad(Ld0_rows, (0, pad))
  ld0_c = jnp.pad(Ld0_cols, (0, pad))
  ld0_v = jnp.pad(Ld0_vals, (0, pad))

  act0 = jnp.pad(X0, ((0, N0P - N0), (0, 0)))
  act1 = X1
  z0 = jnp.zeros((5120 // NTILES, D), F32)
  z1 = jnp.zeros((10240 // NTILES, D), F32)
  for l in (1, 2, 3)[:globals().get("_LAYERS", 3)]:
    w_l0, w_u0, w_d0, w_b0, w_l1, w_u1, w_d1, w_b1 = W[l]
    wc0 = jnp.concatenate([w_l0, w_u0, w_d0, w_b1], axis=1)
    wc1 = jnp.concatenate([w_l1, w_u1, w_d1, w_b0], axis=1)
    y0l, y0u, y0d, y0b = _mm(act0, wc0, tanh=(l > 1))
    y1l, y1u, y1d, y1b = _mm(act1, wc1, tanh=(l > 1))
    h0 = _SC_H0(y0l, y0u, y0d, y1b,
                L0_rows, L0_cols, L0_vals,
                Lu0_rows, Lu0_cols, Lu0_vals,
                ld0_r, ld0_c, ld0_v,
                B1_rows, B1_cols, B1_vals, z0)
    h1 = _SC_H1(y1l, y1u, y1d, y0b,
                L1_rows, L1_cols, L1_vals,
                Lu1_rows, Lu1_cols, Lu1_vals,
                Ld1_rows, Ld1_cols, Ld1_vals,
                B1_cols, B1_rows, B1_vals, z1)
    act0, act1 = h0, h1

  out0 = _mm(act0, fc_W, tanh=True)[0][:N0]
  out1 = _mm(act1, fc_W, tanh=True)[0][:N1]
  return out0, out1
